# speedup vs baseline: 1.0760x; 1.0760x over previous
import functools

import jax
import jax.numpy as jnp
from jax.experimental import pallas as pl
from jax.experimental.pallas import tpu as pltpu

HEADS = 8


def _mm_body(x_ref, w_ref, o_ref):
    o_ref[...] = jnp.dot(x_ref[...], w_ref[...],
                         preferred_element_type=jnp.float32)


def _matmul(x, W, bm=512):
    M, K = x.shape
    _, Nc = W.shape
    Mp = ((M + bm - 1) // bm) * bm
    xp = jnp.pad(x, ((0, Mp - M), (0, 0)))
    out = pl.pallas_call(
        _mm_body,
        grid=(Mp // bm,),
        in_specs=[pl.BlockSpec((bm, K), lambda i: (i, 0)),
                  pl.BlockSpec((K, Nc), lambda i: (0, 0))],
        out_specs=pl.BlockSpec((bm, Nc), lambda i: (i, 0)),
        out_shape=jax.ShapeDtypeStruct((Mp, Nc), jnp.float32),
    )(xp, W)
    return out[:M]


def _gat(x, src, dst, W, a_s, a_d, b, oc, concat):
    N = x.shape[0]
    h = _matmul(x, W).reshape(N, HEADS, oc)
    al_s = jnp.sum(h * a_s[None], axis=-1)
    al_d = jnp.sum(h * a_d[None], axis=-1)
    e = al_s[src] + al_d[dst]
    e = jnp.where(e > 0, e, 0.2 * e)
    ex = jnp.exp(e)
    den = jax.ops.segment_sum(ex, dst, num_segments=N)
    out = jax.ops.segment_sum(h[src] * ex[:, :, None], dst, num_segments=N)
    out = out / (den[:, :, None] + 1e-16)
    if concat:
        out = out.reshape(N, HEADS * oc)
    else:
        out = jnp.mean(out, axis=1)
    return out + b


def kernel(x, edge_index, batch, W0, as0, ad0, b0, W1, as1, ad1, b1, W2, as2,
           ad2, b2, W3, as3, ad3, b3, W4, as4, ad4, b4, linW, linb):
    N = x.shape[0]
    loop = jnp.arange(N, dtype=edge_index.dtype)
    src = jnp.concatenate([edge_index[0], loop])
    dst = jnp.concatenate([edge_index[1], loop])
    h = jax.nn.elu(_gat(x, src, dst, W0, as0, ad0, b0, 64, True))
    h = jax.nn.elu(_gat(h, src, dst, W1, as1, ad1, b1, 256, True))
    h = jax.nn.elu(_gat(h, src, dst, W2, as2, ad2, b2, 256, True))
    h = jax.nn.elu(_gat(h, src, dst, W3, as3, ad3, b3, 256, True))
    h = jax.nn.elu(_gat(h, src, dst, W4, as4, ad4, b4, 256, False))
    G = 64
    s = jax.ops.segment_sum(h, batch, num_segments=G)
    cnt = jax.ops.segment_sum(jnp.ones((N,), dtype=h.dtype), batch,
                              num_segments=G)
    h = s / jnp.maximum(cnt, 1.0)[:, None]
    h = jax.nn.elu(h)
    return h @ linW + linb


# speedup vs baseline: 4.8715x; 4.5272x over previous
import functools

import jax
import jax.numpy as jnp
from jax import lax
from jax.experimental import pallas as pl
from jax.experimental.pallas import tpu as pltpu
from jax.experimental.pallas import tpu_sc as plsc

HEADS = 8
NW = 32
NPW = 320
CHB = 32
N_NODES = 10000


def _mm_body(x_ref, w_ref, o_ref):
    o_ref[...] = jnp.dot(x_ref[...], w_ref[...],
                         preferred_element_type=jnp.float32)


def _matmul(x, W, bm=512):
    M, K = x.shape
    _, Nc = W.shape
    Mp = ((M + bm - 1) // bm) * bm
    xp = jnp.pad(x, ((0, Mp - M), (0, 0)))
    out = pl.pallas_call(
        _mm_body,
        grid=(Mp // bm,),
        in_specs=[pl.BlockSpec((bm, K), lambda i: (i, 0)),
                  pl.BlockSpec((K, Nc), lambda i: (0, 0))],
        out_specs=pl.BlockSpec((bm, Nc), lambda i: (i, 0)),
        out_shape=jax.ShapeDtypeStruct((Mp, Nc), jnp.float32),
    )(xp, W)
    return out[:M]


@functools.partial(jax.jit, static_argnums=(4, 5))
def _agg_sc(h, srcs, exs16, rp_pad, oc, mean):
    N = N_NODES
    HOC = HEADS * oc
    VPH = oc // 16
    NV = HOC // 16
    out_cols = oc if mean else HOC
    OV = out_cols // 16
    mesh = plsc.VectorSubcoreMesh(core_axis_name="c", subcore_axis_name="s")

    @functools.partial(
        pl.kernel, mesh=mesh,
        out_type=jax.ShapeDtypeStruct((N, out_cols), jnp.float32),
        scratch_types=[
            pltpu.VMEM((NPW + 24,), jnp.int32),
            pltpu.VMEM((CHB,), jnp.int32),
            pltpu.VMEM((CHB, HOC), jnp.float32),
            pltpu.VMEM((CHB, 16), jnp.float32),
            pltpu.VMEM((HOC,), jnp.float32),
            pltpu.VMEM((16,), jnp.float32),
            pltpu.VMEM((out_cols,), jnp.float32),
            pltpu.SemaphoreType.DMA,
        ])
    def k(h_hbm, srcs_hbm, ex_hbm, rp_hbm, out_hbm,
          rp_v, idx_v, rows_v, ex_v, acc, den_v, obuf, sem):
        wid = lax.axis_index("s") * 2 + lax.axis_index("c")
        nb = wid * NPW
        pltpu.sync_copy(rp_hbm.at[pl.ds(pl.multiple_of(nb, 8), NPW + 24)],
                        rp_v)

        def rp_at(i):
            return rp_v[pl.ds(i, 16)][0]

        e_start = rp_at(0)
        e_end = rp_at(NPW)
        zero16 = jnp.zeros((16,), jnp.float32)

        def zero_acc():
            for j in range(NV):
                acc[pl.ds(j * 16, 16)] = zero16

        zero_acc()

        def finalize(node_l):
            node_g = nb + node_l
            rpv = rp_v[pl.ds(node_l, 16)]
            deg = rpv[1] - rpv[0]
            den = den_v[pl.ds(0, 16)]
            inv = 1.0 / (den + 1e-16)

            @pl.when(deg > 0)
            def _():
                if mean:
                    for j in range(VPH):
                        s = zero16
                        for hh in range(HEADS):
                            s = s + acc[pl.ds((hh * VPH + j) * 16, 16)] \
                                * inv[hh]
                        obuf[pl.ds(j * 16, 16)] = s * (1.0 / HEADS)
                else:
                    for hh in range(HEADS):
                        inv_s = inv[hh]
                        for j in range(VPH):
                            c0 = (hh * VPH + j) * 16
                            obuf[pl.ds(c0, 16)] = acc[pl.ds(c0, 16)] * inv_s
                pltpu.sync_copy(obuf, out_hbm.at[node_g])
                zero_acc()
                den_v[pl.ds(0, 16)] = zero16

        den_v[pl.ds(0, 16)] = zero16
        ab0 = pl.multiple_of((e_start // 8) * 8, 8)
        nchunks = (e_end - ab0 + CHB - 1) // CHB

        def chunk_body(kk, cur):
            gbase = pl.multiple_of(ab0 + kk * CHB, 8)
            pltpu.sync_copy(srcs_hbm.at[pl.ds(gbase, CHB)], idx_v)
            pltpu.sync_copy(ex_hbm.at[pl.ds(gbase, CHB)], ex_v)
            pltpu.async_copy(h_hbm.at[idx_v], rows_v, sem).wait()
            lo = jnp.maximum(e_start - gbase, 0)
            hi = jnp.minimum(e_end - gbase, CHB)

            def edge_body(li, cur):
                gi = gbase + li
                seg_done = rp_at(cur + 1) <= gi

                @pl.when(seg_done)
                def _():
                    finalize(cur)

                cur = jnp.where(seg_done, cur + 1, cur)
                exrow = ex_v[li]
                plsc.addupdate(den_v.at[pl.ds(0, 16)], exrow)
                for hh in range(HEADS):
                    ex_s = exrow[hh]
                    for j in range(VPH):
                        c0 = (hh * VPH + j) * 16
                        plsc.addupdate(acc.at[pl.ds(c0, 16)],
                                       ex_s * rows_v[li, pl.ds(c0, 16)])
                return cur

            return lax.fori_loop(lo, hi, edge_body, cur)

        cur = lax.fori_loop(0, nchunks, chunk_body, 0)

        def dr_body(i, cur):
            @pl.when(i >= cur)
            def _():
                finalize(i)
            return cur

        lax.fori_loop(0, NPW, dr_body, cur)

    return k(h, srcs, exs16, rp_pad)


def _gat(x, srcs, dsts, rp_pad, W, a_s, a_d, b, oc, concat):
    N = x.shape[0]
    h = _matmul(x, W)
    h3 = h.reshape(N, HEADS, oc)
    al_s = jnp.sum(h3 * a_s[None], axis=-1)
    al_d = jnp.sum(h3 * a_d[None], axis=-1)
    e = al_s[srcs] + al_d[dsts]
    e = jnp.where(e > 0, e, 0.2 * e)
    ex = jnp.exp(e)
    Epad = srcs.shape[0] + 64
    exs16 = jnp.zeros((Epad, 16), jnp.float32).at[:srcs.shape[0], :8].set(ex)
    srcs_pad = jnp.pad(srcs, (0, 64))
    out = _agg_sc(h, srcs_pad, exs16, rp_pad, oc, not concat)
    return out + b


def kernel(x, edge_index, batch, W0, as0, ad0, b0, W1, as1, ad1, b1, W2, as2,
           ad2, b2, W3, as3, ad3, b3, W4, as4, ad4, b4, linW, linb):
    N = x.shape[0]
    loop = jnp.arange(N, dtype=edge_index.dtype)
    src = jnp.concatenate([edge_index[0], loop])
    dst = jnp.concatenate([edge_index[1], loop])
    perm = jnp.argsort(dst)
    srcs = src[perm]
    dsts = dst[perm]
    RP = NW * NPW + NPW + 8
    rp_pad = jnp.searchsorted(dsts, jnp.arange(RP, dtype=jnp.int32)
                              ).astype(jnp.int32)
    h = jax.nn.elu(_gat(x, srcs, dsts, rp_pad, W0, as0, ad0, b0, 64, True))
    h = jax.nn.elu(_gat(h, srcs, dsts, rp_pad, W1, as1, ad1, b1, 256, True))
    h = jax.nn.elu(_gat(h, srcs, dsts, rp_pad, W2, as2, ad2, b2, 256, True))
    h = jax.nn.elu(_gat(h, srcs, dsts, rp_pad, W3, as3, ad3, b3, 256, True))
    h = jax.nn.elu(_gat(h, srcs, dsts, rp_pad, W4, as4, ad4, b4, 256, False))
    G = 64
    s = jax.ops.segment_sum(h, batch, num_segments=G)
    cnt = jax.ops.segment_sum(jnp.ones((N,), dtype=h.dtype), batch,
                              num_segments=G)
    h = s / jnp.maximum(cnt, 1.0)[:, None]
    h = jax.nn.elu(h)
    return h @ linW + linb


# speedup vs baseline: 8.1217x; 1.6672x over previous
import functools

import jax
import jax.numpy as jnp
from jax import lax
from jax.experimental import pallas as pl
from jax.experimental.pallas import tpu as pltpu
from jax.experimental.pallas import tpu_sc as plsc

HEADS = 8
NW = 32
NPW = 320
CHB = 24
N_NODES = 10000


def _mm_body(x_ref, w_ref, o_ref):
    o_ref[...] = jnp.dot(x_ref[...], w_ref[...],
                         preferred_element_type=jnp.float32)


def _matmul(x, W, bm=512):
    M, K = x.shape
    _, Nc = W.shape
    Mp = ((M + bm - 1) // bm) * bm
    xp = jnp.pad(x, ((0, Mp - M), (0, 0)))
    out = pl.pallas_call(
        _mm_body,
        grid=(Mp // bm,),
        in_specs=[pl.BlockSpec((bm, K), lambda i: (i, 0)),
                  pl.BlockSpec((K, Nc), lambda i: (0, 0))],
        out_specs=pl.BlockSpec((bm, Nc), lambda i: (i, 0)),
        out_shape=jax.ShapeDtypeStruct((Mp, Nc), jnp.float32),
    )(xp, W)
    return out[:M]


@functools.partial(jax.jit, static_argnums=(5, 6))
def _agg_sc(h, srcs, exd16, exb128, rp_pad, oc, mean):
    N = N_NODES
    HOC = HEADS * oc
    VPH = oc // 16
    NV = HOC // 16
    out_cols = oc if mean else HOC
    OV = out_cols // 16
    mesh = plsc.VectorSubcoreMesh(core_axis_name="c", subcore_axis_name="s")

    @functools.partial(
        pl.kernel, mesh=mesh,
        out_type=jax.ShapeDtypeStruct((N, out_cols), jnp.float32),
        scratch_types=[
            pltpu.VMEM((NPW + 24,), jnp.int32),
            pltpu.VMEM((CHB,), jnp.int32),
            pltpu.VMEM((CHB,), jnp.int32),
            pltpu.VMEM((CHB, HOC), jnp.float32),
            pltpu.VMEM((CHB, HOC), jnp.float32),
            pltpu.VMEM((CHB, 16), jnp.float32),
            pltpu.VMEM((CHB, 16), jnp.float32),
            pltpu.VMEM((CHB, 128), jnp.float32),
            pltpu.VMEM((CHB, 128), jnp.float32),
            pltpu.VMEM((HOC,), jnp.float32),
            pltpu.VMEM((16,), jnp.float32),
            pltpu.VMEM((out_cols,), jnp.float32),
            pltpu.SemaphoreType.DMA,
            pltpu.SemaphoreType.DMA,
        ])
    def k(h_hbm, srcs_hbm, exd_hbm, exb_hbm, rp_hbm, out_hbm,
          rp_v, idx_v0, idx_v1, rows_v0, rows_v1, exd_v0, exd_v1,
          exb_v0, exb_v1, acc, den_v, obuf, sem0, sem1):
        wid = lax.axis_index("s") * 2 + lax.axis_index("c")
        nb = wid * NPW
        pltpu.sync_copy(rp_hbm.at[pl.ds(pl.multiple_of(nb, 8), NPW + 24)],
                        rp_v)

        def rp_at(i):
            return rp_v[pl.ds(i, 16)][0]

        e_start = rp_at(0)
        e_end = rp_at(NPW)
        zero16 = jnp.zeros((16,), jnp.float32)

        def zero_acc():
            for j in range(NV):
                acc[pl.ds(j * 16, 16)] = zero16

        zero_acc()

        def finalize(node_l):
            node_g = nb + node_l
            rpv = rp_v[pl.ds(node_l, 16)]
            deg = rpv[1] - rpv[0]
            den = den_v[pl.ds(0, 16)]
            inv = 1.0 / (den + 1e-16)

            @pl.when(deg > 0)
            def _():
                if mean:
                    for j in range(VPH):
                        s = zero16
                        for hh in range(HEADS):
                            s = s + acc[pl.ds((hh * VPH + j) * 16, 16)] \
                                * inv[hh]
                        obuf[pl.ds(j * 16, 16)] = s * (1.0 / HEADS)
                else:
                    for hh in range(HEADS):
                        inv_s = inv[hh]
                        for j in range(VPH):
                            c0 = (hh * VPH + j) * 16
                            obuf[pl.ds(c0, 16)] = acc[pl.ds(c0, 16)] * inv_s
                pltpu.sync_copy(obuf, out_hbm.at[node_g])
                zero_acc()
                den_v[pl.ds(0, 16)] = zero16

        den_v[pl.ds(0, 16)] = zero16
        ab0 = pl.multiple_of((e_start // 8) * 8, 8)
        nchunks = (e_end - ab0 + CHB - 1) // CHB
        bufs = ((idx_v0, rows_v0, exd_v0, exb_v0, sem0),
                (idx_v1, rows_v1, exd_v1, exb_v1, sem1))

        def fetch(kk, buf):
            idx_v, rows_v, exd_v, exb_v, sem = buf
            gbase = pl.multiple_of(ab0 + kk * CHB, 8)
            pltpu.sync_copy(srcs_hbm.at[pl.ds(gbase, CHB)], idx_v)
            pltpu.sync_copy(exd_hbm.at[pl.ds(gbase, CHB)], exd_v)
            pltpu.sync_copy(exb_hbm.at[pl.ds(gbase, CHB)], exb_v)
            pltpu.async_copy(h_hbm.at[idx_v], rows_v, sem)

        def process(kk, cur, buf):
            idx_v, rows_v, exd_v, exb_v, sem = buf
            pltpu.make_async_copy(h_hbm.at[idx_v], rows_v, sem).wait()
            gbase = ab0 + kk * CHB
            lo = jnp.maximum(e_start - gbase, 0)
            hi = jnp.minimum(e_end - gbase, CHB)

            def edge_body(li, cur):
                gi = gbase + li
                seg_done = rp_at(cur + 1) <= gi

                @pl.when(seg_done)
                def _():
                    finalize(cur)

                cur = jnp.where(seg_done, cur + 1, cur)
                plsc.addupdate(den_v.at[pl.ds(0, 16)], exd_v[li])

                @plsc.parallel_loop(0, NV, unroll=8)
                def _(i):
                    c0 = i * 16
                    hb = (i // VPH) * 16
                    plsc.addupdate(
                        acc.at[pl.ds(c0, 16)],
                        exb_v[li, pl.ds(hb, 16)]
                        * rows_v[li, pl.ds(c0, 16)])

                return cur

            return lax.fori_loop(lo, hi, edge_body, cur)

        fetch(0, bufs[0])
        fetch(1, bufs[1])
        npairs = (nchunks + 1) // 2

        def pair_body(t, cur):
            cur = process(2 * t, cur, bufs[0])
            fetch(2 * t + 2, bufs[0])
            cur = process(2 * t + 1, cur, bufs[1])
            fetch(2 * t + 3, bufs[1])
            return cur

        cur = lax.fori_loop(0, npairs, pair_body, 0)
        for b in bufs:
            pltpu.make_async_copy(h_hbm.at[b[0]], b[1], b[4]).wait()

        def dr_body(i, cur):
            @pl.when(i >= cur)
            def _():
                finalize(i)
            return cur

        lax.fori_loop(0, NPW, dr_body, cur)

    return k(h, srcs, exd16, exb128, rp_pad)


def _gat(x, srcs, dsts, rp_pad, W, a_s, a_d, b, oc, concat):
    N = x.shape[0]
    h = _matmul(x, W)
    h3 = h.reshape(N, HEADS, oc)
    al_s = jnp.sum(h3 * a_s[None], axis=-1)
    al_d = jnp.sum(h3 * a_d[None], axis=-1)
    e = al_s[srcs] + al_d[dsts]
    e = jnp.where(e > 0, e, 0.2 * e)
    ex = jnp.exp(e)
    E = srcs.shape[0]
    Epad = E + 112
    exd16 = jnp.zeros((Epad, 16), jnp.float32).at[:E, :8].set(ex)
    exb128 = jnp.zeros((Epad, 128), jnp.float32).at[:E].set(
        jnp.repeat(ex, 16, axis=1))
    srcs_pad = jnp.pad(srcs, (0, 112))
    out = _agg_sc(h, srcs_pad, exd16, exb128, rp_pad, oc, not concat)
    return out + b


def kernel(x, edge_index, batch, W0, as0, ad0, b0, W1, as1, ad1, b1, W2, as2,
           ad2, b2, W3, as3, ad3, b3, W4, as4, ad4, b4, linW, linb):
    N = x.shape[0]
    loop = jnp.arange(N, dtype=edge_index.dtype)
    src = jnp.concatenate([edge_index[0], loop])
    dst = jnp.concatenate([edge_index[1], loop])
    perm = jnp.argsort(dst)
    srcs = src[perm]
    dsts = dst[perm]
    RP = NW * NPW + NPW + 8
    rp_pad = jnp.searchsorted(dsts, jnp.arange(RP, dtype=jnp.int32)
                              ).astype(jnp.int32)
    h = jax.nn.elu(_gat(x, srcs, dsts, rp_pad, W0, as0, ad0, b0, 64, True))
    h = jax.nn.elu(_gat(h, srcs, dsts, rp_pad, W1, as1, ad1, b1, 256, True))
    h = jax.nn.elu(_gat(h, srcs, dsts, rp_pad, W2, as2, ad2, b2, 256, True))
    h = jax.nn.elu(_gat(h, srcs, dsts, rp_pad, W3, as3, ad3, b3, 256, True))
    h = jax.nn.elu(_gat(h, srcs, dsts, rp_pad, W4, as4, ad4, b4, 256, False))
    G = 64
    s = jax.ops.segment_sum(h, batch, num_segments=G)
    cnt = jax.ops.segment_sum(jnp.ones((N,), dtype=h.dtype), batch,
                              num_segments=G)
    h = s / jnp.maximum(cnt, 1.0)[:, None]
    h = jax.nn.elu(h)
    return h @ linW + linb


# speedup vs baseline: 11.2403x; 1.3840x over previous
import functools

import jax
import jax.numpy as jnp
from jax import lax
from jax.experimental import pallas as pl
from jax.experimental.pallas import tpu as pltpu
from jax.experimental.pallas import tpu_sc as plsc

HEADS = 8
NW = 32
NPW = 320
CHB = 24
N_NODES = 10000


def _mm_body(x_ref, w_ref, o_ref):
    o_ref[...] = jnp.dot(x_ref[...], w_ref[...],
                         preferred_element_type=jnp.float32)


def _matmul(x, W, bm=512):
    M, K = x.shape
    _, Nc = W.shape
    Mp = ((M + bm - 1) // bm) * bm
    xp = jnp.pad(x, ((0, Mp - M), (0, 0)))
    out = pl.pallas_call(
        _mm_body,
        grid=(Mp // bm,),
        in_specs=[pl.BlockSpec((bm, K), lambda i: (i, 0)),
                  pl.BlockSpec((K, Nc), lambda i: (0, 0))],
        out_specs=pl.BlockSpec((bm, Nc), lambda i: (i, 0)),
        out_shape=jax.ShapeDtypeStruct((Mp, Nc), jnp.float32),
    )(xp, W)
    return out[:M]


@functools.partial(jax.jit, static_argnums=(5, 6))
def _agg_sc(h, srcs, als16, ald16, rp_pad, oc, mean):
    N = N_NODES
    HOC = HEADS * oc
    VPH = oc // 16
    NV = HOC // 16
    out_cols = oc if mean else HOC
    OV = out_cols // 16
    mesh = plsc.VectorSubcoreMesh(core_axis_name="c", subcore_axis_name="s")

    @functools.partial(
        pl.kernel, mesh=mesh,
        out_type=jax.ShapeDtypeStruct((N, out_cols), jnp.float32),
        scratch_types=[
            pltpu.VMEM((NPW + 24,), jnp.int32),
            pltpu.VMEM((CHB,), jnp.int32),
            pltpu.VMEM((CHB,), jnp.int32),
            pltpu.VMEM((CHB, HOC), jnp.float32),
            pltpu.VMEM((CHB, HOC), jnp.float32),
            pltpu.VMEM((CHB, 128), jnp.float32),
            pltpu.VMEM((CHB, 128), jnp.float32),
            pltpu.VMEM((NPW * 16,), jnp.float32),
            pltpu.VMEM((128,), jnp.float32),
            pltpu.VMEM((128,), jnp.float32),
            pltpu.VMEM((HOC,), jnp.float32),
            pltpu.VMEM((128,), jnp.float32),
            pltpu.VMEM((out_cols,), jnp.float32),
            pltpu.SemaphoreType.DMA,
            pltpu.SemaphoreType.DMA,
            pltpu.SemaphoreType.DMA,
            pltpu.SemaphoreType.DMA,
        ])
    def k(h_hbm, srcs_hbm, als_hbm, ald_hbm, rp_hbm, out_hbm,
          rp_v, idx_v0, idx_v1, rows_v0, rows_v1, als_v0, als_v1,
          ald_v, ald_rep, exb_sc, acc, den_v, obuf,
          sem0, sem1, sem0b, sem1b):
        wid = lax.axis_index("s") * 2 + lax.axis_index("c")
        nb = wid * NPW
        pltpu.sync_copy(rp_hbm.at[pl.ds(pl.multiple_of(nb, 8), NPW + 24)],
                        rp_v)
        pltpu.sync_copy(
            ald_hbm.at[pl.ds(pl.multiple_of(nb * 16, 8), NPW * 16)], ald_v)

        def rp_at(i):
            return rp_v[pl.ds(i, 16)][0]

        e_start = rp_at(0)
        e_end = rp_at(NPW)
        zero16 = jnp.zeros((16,), jnp.float32)

        def zero_acc():
            for j in range(NV):
                acc[pl.ds(j * 16, 16)] = zero16

        zero_acc()

        def finalize(node_l):
            node_g = nb + node_l
            rpv = rp_v[pl.ds(node_l, 16)]
            deg = rpv[1] - rpv[0]

            @pl.when(deg > 0)
            def _():
                if mean:
                    for j in range(VPH):
                        s = zero16
                        for hh in range(HEADS):
                            invb = 1.0 / (den_v[pl.ds(hh * 16, 16)] + 1e-16)
                            s = s + acc[pl.ds((hh * VPH + j) * 16, 16)] \
                                * invb
                        obuf[pl.ds(j * 16, 16)] = s * (1.0 / HEADS)
                else:
                    for hh in range(HEADS):
                        invb = 1.0 / (den_v[pl.ds(hh * 16, 16)] + 1e-16)
                        for j in range(VPH):
                            c0 = (hh * VPH + j) * 16
                            obuf[pl.ds(c0, 16)] = acc[pl.ds(c0, 16)] * invb
                pltpu.sync_copy(obuf, out_hbm.at[node_g])
                zero_acc()
                for hh in range(HEADS):
                    den_v[pl.ds(hh * 16, 16)] = zero16

        for hh in range(HEADS):
            den_v[pl.ds(hh * 16, 16)] = zero16
        nxt0 = ald_v[pl.ds(0, 16)]
        for hh in range(HEADS):
            ald_rep[pl.ds(hh * 16, 16)] = nxt0[hh] + zero16
        ab0 = pl.multiple_of((e_start // 8) * 8, 8)
        nchunks = (e_end - ab0 + CHB - 1) // CHB
        bufs = ((idx_v0, rows_v0, als_v0, sem0, sem0b),
                (idx_v1, rows_v1, als_v1, sem1, sem1b))

        def fetch(kk, buf):
            idx_v, rows_v, als_v, sem, semb = buf
            gbase = pl.multiple_of(ab0 + kk * CHB, 8)
            pltpu.sync_copy(srcs_hbm.at[pl.ds(gbase, CHB)], idx_v)
            pltpu.async_copy(h_hbm.at[idx_v], rows_v, sem)
            pltpu.async_copy(als_hbm.at[idx_v], als_v, semb)

        def process(kk, cur, buf):
            idx_v, rows_v, als_v, sem, semb = buf
            pltpu.make_async_copy(h_hbm.at[idx_v], rows_v, sem).wait()
            pltpu.make_async_copy(als_hbm.at[idx_v], als_v, semb).wait()
            gbase = ab0 + kk * CHB
            lo = jnp.maximum(e_start - gbase, 0)
            hi = jnp.minimum(e_end - gbase, CHB)

            def edge_body(li, cur):
                gi = gbase + li
                seg_done = rp_at(cur + 1) <= gi

                @pl.when(seg_done)
                def _():
                    finalize(cur)
                    nxt = ald_v[pl.ds((cur + 1) * 16, 16)]
                    for hh in range(HEADS):
                        ald_rep[pl.ds(hh * 16, 16)] = nxt[hh] + zero16

                cur = jnp.where(seg_done, cur + 1, cur)
                for hh in range(HEADS):
                    hb = hh * 16
                    e = als_v[li, pl.ds(hb, 16)] + ald_rep[pl.ds(hb, 16)]
                    ex = jnp.exp(jnp.maximum(e, 0.2 * e))
                    plsc.addupdate(den_v.at[pl.ds(hb, 16)], ex)
                    exb_sc[pl.ds(hb, 16)] = ex

                @plsc.parallel_loop(0, NV, unroll=8)
                def _(i):
                    c0 = i * 16
                    hb = (i // VPH) * 16
                    plsc.addupdate(
                        acc.at[pl.ds(c0, 16)],
                        exb_sc[pl.ds(hb, 16)]
                        * rows_v[li, pl.ds(c0, 16)])

                return cur

            return lax.fori_loop(lo, hi, edge_body, cur)

        fetch(0, bufs[0])
        fetch(1, bufs[1])
        npairs = (nchunks + 1) // 2

        def pair_body(t, cur):
            cur = process(2 * t, cur, bufs[0])
            fetch(2 * t + 2, bufs[0])
            cur = process(2 * t + 1, cur, bufs[1])
            fetch(2 * t + 3, bufs[1])
            return cur

        cur = lax.fori_loop(0, npairs, pair_body, 0)
        for b in bufs:
            pltpu.make_async_copy(h_hbm.at[b[0]], b[1], b[3]).wait()
            pltpu.make_async_copy(als_hbm.at[b[0]], b[2], b[4]).wait()

        def dr_body(i, cur):
            @pl.when(i >= cur)
            def _():
                finalize(i)
            return cur

        lax.fori_loop(0, NPW, dr_body, cur)

    return k(h, srcs, als16, ald16, rp_pad)


def _gat(x, srcs_pad, rp_pad, W, a_s, a_d, b, oc, concat):
    N = x.shape[0]
    h = _matmul(x, W)
    h3 = h.reshape(N, HEADS, oc)
    al_s = jnp.sum(h3 * a_s[None], axis=-1)
    al_d = jnp.sum(h3 * a_d[None], axis=-1)
    als128 = jnp.repeat(al_s, 16, axis=1)
    ald16 = jnp.zeros((NW * NPW, 16), jnp.float32).at[:N, :8].set(
        al_d).reshape(-1)
    out = _agg_sc(h, srcs_pad, als128, ald16, rp_pad, oc, not concat)
    return out + b


def kernel(x, edge_index, batch, W0, as0, ad0, b0, W1, as1, ad1, b1, W2, as2,
           ad2, b2, W3, as3, ad3, b3, W4, as4, ad4, b4, linW, linb):
    N = x.shape[0]
    loop = jnp.arange(N, dtype=edge_index.dtype)
    src = jnp.concatenate([edge_index[0], loop])
    dst = jnp.concatenate([edge_index[1], loop])
    perm = jnp.argsort(dst)
    srcs = src[perm]
    dsts = dst[perm]
    srcs_pad = jnp.pad(srcs, (0, 112))
    RP = NW * NPW + NPW + 8
    rp_pad = jnp.searchsorted(dsts, jnp.arange(RP, dtype=jnp.int32)
                              ).astype(jnp.int32)
    h = jax.nn.elu(_gat(x, srcs_pad, rp_pad, W0, as0, ad0, b0, 64, True))
    h = jax.nn.elu(_gat(h, srcs_pad, rp_pad, W1, as1, ad1, b1, 256, True))
    h = jax.nn.elu(_gat(h, srcs_pad, rp_pad, W2, as2, ad2, b2, 256, True))
    h = jax.nn.elu(_gat(h, srcs_pad, rp_pad, W3, as3, ad3, b3, 256, True))
    h = jax.nn.elu(_gat(h, srcs_pad, rp_pad, W4, as4, ad4, b4, 256, False))
    G = 64
    s = jax.ops.segment_sum(h, batch, num_segments=G)
    cnt = jax.ops.segment_sum(jnp.ones((N,), dtype=h.dtype), batch,
                              num_segments=G)
    h = s / jnp.maximum(cnt, 1.0)[:, None]
    h = jax.nn.elu(h)
    return h @ linW + linb


# speedup vs baseline: 11.7125x; 1.0420x over previous
import functools

import jax
import jax.numpy as jnp
from jax import lax
from jax.experimental import pallas as pl
from jax.experimental.pallas import tpu as pltpu
from jax.experimental.pallas import tpu_sc as plsc

HEADS = 8
NW = 32
NPW = 320
CHB_BIG = 24
CHB_SMALL = 80
N_NODES = 10000


def _mm_body(x_ref, w_ref, o_ref):
    o_ref[...] = jnp.dot(x_ref[...], w_ref[...],
                         preferred_element_type=jnp.float32)


def _matmul(x, W, bm=512):
    M, K = x.shape
    _, Nc = W.shape
    Mp = ((M + bm - 1) // bm) * bm
    xp = jnp.pad(x, ((0, Mp - M), (0, 0)))
    out = pl.pallas_call(
        _mm_body,
        grid=(Mp // bm,),
        in_specs=[pl.BlockSpec((bm, K), lambda i: (i, 0)),
                  pl.BlockSpec((K, Nc), lambda i: (0, 0))],
        out_specs=pl.BlockSpec((bm, Nc), lambda i: (i, 0)),
        out_shape=jax.ShapeDtypeStruct((Mp, Nc), jnp.float32),
    )(xp, W)
    return out[:M]


@functools.partial(jax.jit, static_argnums=(5, 6))
def _agg_sc(h, srcs, als16, ald16, rp_pad, oc, mean):
    N = N_NODES
    CHB = CHB_SMALL if oc == 64 else CHB_BIG
    HOC = HEADS * oc
    VPH = oc // 16
    NV = HOC // 16
    out_cols = oc if mean else HOC
    OV = out_cols // 16
    mesh = plsc.VectorSubcoreMesh(core_axis_name="c", subcore_axis_name="s")

    @functools.partial(
        pl.kernel, mesh=mesh,
        out_type=jax.ShapeDtypeStruct((N, out_cols), jnp.float32),
        scratch_types=[
            pltpu.VMEM((NPW + 24,), jnp.int32),
            pltpu.VMEM((CHB,), jnp.int32),
            pltpu.VMEM((CHB,), jnp.int32),
            pltpu.VMEM((CHB, HOC), jnp.float32),
            pltpu.VMEM((CHB, HOC), jnp.float32),
            pltpu.VMEM((CHB, 128), jnp.float32),
            pltpu.VMEM((CHB, 128), jnp.float32),
            pltpu.VMEM((NPW * 16,), jnp.float32),
            pltpu.VMEM((128,), jnp.float32),
            pltpu.VMEM((128,), jnp.float32),
            pltpu.VMEM((HOC,), jnp.float32),
            pltpu.VMEM((128,), jnp.float32),
            pltpu.VMEM((out_cols,), jnp.float32),
            pltpu.SemaphoreType.DMA,
            pltpu.SemaphoreType.DMA,
            pltpu.SemaphoreType.DMA,
            pltpu.SemaphoreType.DMA,
            pltpu.SemaphoreType.DMA,
            pltpu.SemaphoreType.DMA,
        ])
    def k(h_hbm, srcs_hbm, als_hbm, ald_hbm, rp_hbm, out_hbm,
          rp_v, idx_v0, idx_v1, rows_v0, rows_v1, als_v0, als_v1,
          ald_v, ald_rep, exb_sc, acc, den_v, obuf,
          sem0, sem1, sem0b, sem1b, semi0, semi1):
        wid = lax.axis_index("s") * 2 + lax.axis_index("c")
        nb = wid * NPW
        pltpu.sync_copy(rp_hbm.at[pl.ds(pl.multiple_of(nb, 8), NPW + 24)],
                        rp_v)
        pltpu.sync_copy(
            ald_hbm.at[pl.ds(pl.multiple_of(nb * 16, 8), NPW * 16)], ald_v)

        def rp_at(i):
            return rp_v[pl.ds(i, 16)][0]

        e_start = rp_at(0)
        e_end = rp_at(NPW)
        zero16 = jnp.zeros((16,), jnp.float32)

        def zero_acc():
            for j in range(NV):
                acc[pl.ds(j * 16, 16)] = zero16

        zero_acc()

        def finalize(node_l):
            node_g = nb + node_l
            rpv = rp_v[pl.ds(node_l, 16)]
            deg = rpv[1] - rpv[0]

            @pl.when(deg > 0)
            def _():
                if mean:
                    for j in range(VPH):
                        s = zero16
                        for hh in range(HEADS):
                            invb = 1.0 / (den_v[pl.ds(hh * 16, 16)] + 1e-16)
                            s = s + acc[pl.ds((hh * VPH + j) * 16, 16)] \
                                * invb
                        obuf[pl.ds(j * 16, 16)] = s * (1.0 / HEADS)
                else:
                    for hh in range(HEADS):
                        invb = 1.0 / (den_v[pl.ds(hh * 16, 16)] + 1e-16)
                        for j in range(VPH):
                            c0 = (hh * VPH + j) * 16
                            obuf[pl.ds(c0, 16)] = acc[pl.ds(c0, 16)] * invb
                pltpu.sync_copy(obuf, out_hbm.at[node_g])
                zero_acc()
                for hh in range(HEADS):
                    den_v[pl.ds(hh * 16, 16)] = zero16

        for hh in range(HEADS):
            den_v[pl.ds(hh * 16, 16)] = zero16
        nxt0 = ald_v[pl.ds(0, 16)]
        for hh in range(HEADS):
            ald_rep[pl.ds(hh * 16, 16)] = nxt0[hh] + zero16
        ab0 = pl.multiple_of((e_start // 8) * 8, 8)
        nchunks = (e_end - ab0 + CHB - 1) // CHB
        bufs = ((idx_v0, rows_v0, als_v0, sem0, sem0b, semi0),
                (idx_v1, rows_v1, als_v1, sem1, sem1b, semi1))

        def fetch_idx(kk, buf):
            idx_v, rows_v, als_v, sem, semb, semi = buf
            gbase = pl.multiple_of(ab0 + kk * CHB, 8)
            pltpu.async_copy(srcs_hbm.at[pl.ds(gbase, CHB)], idx_v, semi)

        def fetch_gather(kk, buf):
            idx_v, rows_v, als_v, sem, semb, semi = buf
            gbase = pl.multiple_of(ab0 + kk * CHB, 8)
            pltpu.make_async_copy(srcs_hbm.at[pl.ds(gbase, CHB)], idx_v,
                                  semi).wait()
            pltpu.async_copy(h_hbm.at[idx_v], rows_v, sem)
            pltpu.async_copy(als_hbm.at[idx_v], als_v, semb)

        def process(kk, cur, buf):
            idx_v, rows_v, als_v, sem, semb, semi = buf
            pltpu.make_async_copy(h_hbm.at[idx_v], rows_v, sem).wait()
            pltpu.make_async_copy(als_hbm.at[idx_v], als_v, semb).wait()
            fetch_idx(kk + 2, buf)
            gbase = ab0 + kk * CHB
            lo = jnp.maximum(e_start - gbase, 0)
            hi = jnp.minimum(e_end - gbase, CHB)

            def edge_body(li, cur):
                gi = gbase + li
                seg_done = rp_at(cur + 1) <= gi

                @pl.when(seg_done)
                def _():
                    finalize(cur)
                    nxt = ald_v[pl.ds((cur + 1) * 16, 16)]
                    for hh in range(HEADS):
                        ald_rep[pl.ds(hh * 16, 16)] = nxt[hh] + zero16

                cur = jnp.where(seg_done, cur + 1, cur)
                for hh in range(HEADS):
                    hb = hh * 16
                    e = als_v[li, pl.ds(hb, 16)] + ald_rep[pl.ds(hb, 16)]
                    ex = jnp.exp(jnp.maximum(e, 0.2 * e))
                    plsc.addupdate(den_v.at[pl.ds(hb, 16)], ex)
                    exb_sc[pl.ds(hb, 16)] = ex

                @plsc.parallel_loop(0, NV, unroll=8)
                def _(i):
                    c0 = i * 16
                    hb = (i // VPH) * 16
                    plsc.addupdate(
                        acc.at[pl.ds(c0, 16)],
                        exb_sc[pl.ds(hb, 16)]
                        * rows_v[li, pl.ds(c0, 16)])

                return cur

            return lax.fori_loop(lo, hi, edge_body, cur)

        fetch_idx(0, bufs[0])
        fetch_gather(0, bufs[0])
        fetch_idx(1, bufs[1])
        fetch_gather(1, bufs[1])
        npairs = (nchunks + 1) // 2

        def pair_body(t, cur):
            cur = process(2 * t, cur, bufs[0])
            fetch_gather(2 * t + 2, bufs[0])
            cur = process(2 * t + 1, cur, bufs[1])
            fetch_gather(2 * t + 3, bufs[1])
            return cur

        cur = lax.fori_loop(0, npairs, pair_body, 0)
        for b in bufs:
            pltpu.make_async_copy(h_hbm.at[b[0]], b[1], b[3]).wait()
            pltpu.make_async_copy(als_hbm.at[b[0]], b[2], b[4]).wait()

        def dr_body(i, cur):
            @pl.when(i >= cur)
            def _():
                finalize(i)
            return cur

        lax.fori_loop(0, NPW, dr_body, cur)

    return k(h, srcs, als16, ald16, rp_pad)


def _gat(x, srcs_pad, rp_pad, W, a_s, a_d, b, oc, concat):
    N = x.shape[0]
    h = _matmul(x, W)
    h3 = h.reshape(N, HEADS, oc)
    al_s = jnp.sum(h3 * a_s[None], axis=-1)
    al_d = jnp.sum(h3 * a_d[None], axis=-1)
    als128 = jnp.repeat(al_s, 16, axis=1)
    ald16 = jnp.zeros((NW * NPW, 16), jnp.float32).at[:N, :8].set(
        al_d).reshape(-1)
    out = _agg_sc(h, srcs_pad, als128, ald16, rp_pad, oc, not concat)
    return out + b


def kernel(x, edge_index, batch, W0, as0, ad0, b0, W1, as1, ad1, b1, W2, as2,
           ad2, b2, W3, as3, ad3, b3, W4, as4, ad4, b4, linW, linb):
    N = x.shape[0]
    loop = jnp.arange(N, dtype=edge_index.dtype)
    src = jnp.concatenate([edge_index[0], loop])
    dst = jnp.concatenate([edge_index[1], loop])
    perm = jnp.argsort(dst)
    srcs = src[perm]
    dsts = dst[perm]
    srcs_pad = jnp.pad(srcs, (0, 336))
    RP = NW * NPW + NPW + 8
    rp_pad = jnp.searchsorted(dsts, jnp.arange(RP, dtype=jnp.int32)
                              ).astype(jnp.int32)
    h = jax.nn.elu(_gat(x, srcs_pad, rp_pad, W0, as0, ad0, b0, 64, True))
    h = jax.nn.elu(_gat(h, srcs_pad, rp_pad, W1, as1, ad1, b1, 256, True))
    h = jax.nn.elu(_gat(h, srcs_pad, rp_pad, W2, as2, ad2, b2, 256, True))
    h = jax.nn.elu(_gat(h, srcs_pad, rp_pad, W3, as3, ad3, b3, 256, True))
    h = jax.nn.elu(_gat(h, srcs_pad, rp_pad, W4, as4, ad4, b4, 256, False))
    G = 64
    s = jax.ops.segment_sum(h, batch, num_segments=G)
    cnt = jax.ops.segment_sum(jnp.ones((N,), dtype=h.dtype), batch,
                              num_segments=G)
    h = s / jnp.maximum(cnt, 1.0)[:, None]
    h = jax.nn.elu(h)
    return h @ linW + linb


# speedup vs baseline: 11.8070x; 1.0081x over previous
import functools

import jax
import jax.numpy as jnp
from jax import lax
from jax.experimental import pallas as pl
from jax.experimental.pallas import tpu as pltpu
from jax.experimental.pallas import tpu_sc as plsc

HEADS = 8
NW = 32
NPW = 320
CHB_BIG = 24
CHB_SMALL = 80
N_NODES = 10000


def _mm_body(x_ref, w_ref, o_ref):
    o_ref[...] = jnp.dot(x_ref[...], w_ref[...],
                         preferred_element_type=jnp.float32)


def _matmul(x, W, bm=512):
    M, K = x.shape
    _, Nc = W.shape
    Mp = ((M + bm - 1) // bm) * bm
    xp = jnp.pad(x, ((0, Mp - M), (0, 0)))
    out = pl.pallas_call(
        _mm_body,
        grid=(Mp // bm,),
        in_specs=[pl.BlockSpec((bm, K), lambda i: (i, 0)),
                  pl.BlockSpec((K, Nc), lambda i: (0, 0))],
        out_specs=pl.BlockSpec((bm, Nc), lambda i: (i, 0)),
        out_shape=jax.ShapeDtypeStruct((Mp, Nc), jnp.float32),
    )(xp, W)
    return out[:M]


@functools.partial(jax.jit, static_argnums=(5, 6))
def _agg_sc(h, srcs, als16, ald16, rp_pad, oc, mean):
    N = N_NODES
    CHB = CHB_SMALL if oc == 64 else CHB_BIG
    HOC = HEADS * oc
    VPH = oc // 16
    NV = HOC // 16
    out_cols = oc if mean else HOC
    OV = out_cols // 16
    mesh = plsc.VectorSubcoreMesh(core_axis_name="c", subcore_axis_name="s")

    @functools.partial(
        pl.kernel, mesh=mesh,
        out_type=jax.ShapeDtypeStruct((N, out_cols), jnp.float32),
        scratch_types=[
            pltpu.VMEM((NPW + 24,), jnp.int32),
            pltpu.VMEM((CHB,), jnp.int32),
            pltpu.VMEM((CHB,), jnp.int32),
            pltpu.VMEM((CHB, HOC), jnp.float32),
            pltpu.VMEM((CHB, HOC), jnp.float32),
            pltpu.VMEM((CHB, 128), jnp.float32),
            pltpu.VMEM((CHB, 128), jnp.float32),
            pltpu.VMEM((NPW * 16,), jnp.float32),
            pltpu.VMEM((128,), jnp.float32),
            pltpu.VMEM((128,), jnp.float32),
            pltpu.VMEM((HOC,), jnp.float32),
            pltpu.VMEM((128,), jnp.float32),
            pltpu.VMEM((out_cols,), jnp.float32),
            pltpu.SemaphoreType.DMA,
            pltpu.SemaphoreType.DMA,
            pltpu.SemaphoreType.DMA,
            pltpu.SemaphoreType.DMA,
            pltpu.SemaphoreType.DMA,
            pltpu.SemaphoreType.DMA,
        ])
    def k(h_hbm, srcs_hbm, als_hbm, ald_hbm, rp_hbm, out_hbm,
          rp_v, idx_v0, idx_v1, rows_v0, rows_v1, als_v0, als_v1,
          ald_v, ald_rep, exb_sc, acc, den_v, obuf,
          sem0, sem1, sem0b, sem1b, semi0, semi1):
        wid = lax.axis_index("s") * 2 + lax.axis_index("c")
        nb = wid * NPW
        pltpu.sync_copy(rp_hbm.at[pl.ds(pl.multiple_of(nb, 8), NPW + 24)],
                        rp_v)
        pltpu.sync_copy(
            ald_hbm.at[pl.ds(pl.multiple_of(nb * 16, 8), NPW * 16)], ald_v)

        def rp_at(i):
            return rp_v[pl.ds(i, 16)][0]

        e_start = rp_at(0)
        e_end = rp_at(NPW)
        zero16 = jnp.zeros((16,), jnp.float32)

        def zero_acc():
            for j in range(NV):
                acc[pl.ds(j * 16, 16)] = zero16

        zero_acc()

        def finalize(node_l):
            node_g = nb + node_l
            rpv = rp_v[pl.ds(node_l, 16)]
            deg = rpv[1] - rpv[0]

            @pl.when(deg > 0)
            def _():
                if mean:
                    for j in range(VPH):
                        s = zero16
                        for hh in range(HEADS):
                            invb = 1.0 / (den_v[pl.ds(hh * 16, 16)] + 1e-16)
                            s = s + acc[pl.ds((hh * VPH + j) * 16, 16)] \
                                * invb
                        obuf[pl.ds(j * 16, 16)] = s * (1.0 / HEADS)
                else:
                    for hh in range(HEADS):
                        invb = 1.0 / (den_v[pl.ds(hh * 16, 16)] + 1e-16)
                        for j in range(VPH):
                            c0 = (hh * VPH + j) * 16
                            obuf[pl.ds(c0, 16)] = acc[pl.ds(c0, 16)] * invb
                pltpu.sync_copy(obuf, out_hbm.at[node_g])
                zero_acc()
                for hh in range(HEADS):
                    den_v[pl.ds(hh * 16, 16)] = zero16

        for hh in range(HEADS):
            den_v[pl.ds(hh * 16, 16)] = zero16
        nxt0 = ald_v[pl.ds(0, 16)]
        for hh in range(HEADS):
            ald_rep[pl.ds(hh * 16, 16)] = nxt0[hh] + zero16
        ab0 = pl.multiple_of((e_start // 8) * 8, 8)
        nchunks = (e_end - ab0 + CHB - 1) // CHB
        bufs = ((idx_v0, rows_v0, als_v0, sem0, sem0b, semi0),
                (idx_v1, rows_v1, als_v1, sem1, sem1b, semi1))

        def fetch_idx(kk, buf):
            idx_v, rows_v, als_v, sem, semb, semi = buf
            gbase = pl.multiple_of(ab0 + kk * CHB, 8)
            pltpu.async_copy(srcs_hbm.at[pl.ds(gbase, CHB)], idx_v, semi)

        def fetch_gather(kk, buf):
            idx_v, rows_v, als_v, sem, semb, semi = buf
            gbase = pl.multiple_of(ab0 + kk * CHB, 8)
            pltpu.make_async_copy(srcs_hbm.at[pl.ds(gbase, CHB)], idx_v,
                                  semi).wait()
            pltpu.async_copy(h_hbm.at[idx_v], rows_v, sem)
            pltpu.async_copy(als_hbm.at[idx_v], als_v, semb)

        def process(kk, cur, buf):
            idx_v, rows_v, als_v, sem, semb, semi = buf
            pltpu.make_async_copy(h_hbm.at[idx_v], rows_v, sem).wait()
            pltpu.make_async_copy(als_hbm.at[idx_v], als_v, semb).wait()
            fetch_idx(kk + 2, buf)
            gbase = ab0 + kk * CHB
            lo = jnp.maximum(e_start - gbase, 0)
            hi = jnp.minimum(e_end - gbase, CHB)

            def edge_body(li, cur):
                gi = gbase + li
                seg_done = rp_at(cur + 1) <= gi

                @pl.when(seg_done)
                def _():
                    finalize(cur)
                    nxt = ald_v[pl.ds((cur + 1) * 16, 16)]
                    for hh in range(HEADS):
                        ald_rep[pl.ds(hh * 16, 16)] = nxt[hh] + zero16

                cur = jnp.where(seg_done, cur + 1, cur)
                for hh in range(HEADS):
                    hb = hh * 16
                    e = als_v[li, pl.ds(hb, 16)] + ald_rep[pl.ds(hb, 16)]
                    ex = jnp.exp(jnp.maximum(e, 0.2 * e))
                    plsc.addupdate(den_v.at[pl.ds(hb, 16)], ex)
                    exb_sc[pl.ds(hb, 16)] = ex

                @plsc.parallel_loop(0, NV, unroll=16)
                def _(i):
                    c0 = i * 16
                    hb = (i // VPH) * 16
                    plsc.addupdate(
                        acc.at[pl.ds(c0, 16)],
                        exb_sc[pl.ds(hb, 16)]
                        * rows_v[li, pl.ds(c0, 16)])

                return cur

            return lax.fori_loop(lo, hi, edge_body, cur)

        fetch_idx(0, bufs[0])
        fetch_gather(0, bufs[0])
        fetch_idx(1, bufs[1])
        fetch_gather(1, bufs[1])
        npairs = (nchunks + 1) // 2

        def pair_body(t, cur):
            cur = process(2 * t, cur, bufs[0])
            fetch_gather(2 * t + 2, bufs[0])
            cur = process(2 * t + 1, cur, bufs[1])
            fetch_gather(2 * t + 3, bufs[1])
            return cur

        cur = lax.fori_loop(0, npairs, pair_body, 0)
        for b in bufs:
            pltpu.make_async_copy(h_hbm.at[b[0]], b[1], b[3]).wait()
            pltpu.make_async_copy(als_hbm.at[b[0]], b[2], b[4]).wait()

        def dr_body(i, cur):
            @pl.when(i >= cur)
            def _():
                finalize(i)
            return cur

        lax.fori_loop(0, NPW, dr_body, cur)

    return k(h, srcs, als16, ald16, rp_pad)


def _gat(x, srcs_pad, rp_pad, W, a_s, a_d, b, oc, concat):
    N = x.shape[0]
    h = _matmul(x, W)
    h3 = h.reshape(N, HEADS, oc)
    al_s = jnp.sum(h3 * a_s[None], axis=-1)
    al_d = jnp.sum(h3 * a_d[None], axis=-1)
    als128 = jnp.repeat(al_s, 16, axis=1)
    ald16 = jnp.zeros((NW * NPW, 16), jnp.float32).at[:N, :8].set(
        al_d).reshape(-1)
    out = _agg_sc(h, srcs_pad, als128, ald16, rp_pad, oc, not concat)
    return out + b


def kernel(x, edge_index, batch, W0, as0, ad0, b0, W1, as1, ad1, b1, W2, as2,
           ad2, b2, W3, as3, ad3, b3, W4, as4, ad4, b4, linW, linb):
    N = x.shape[0]
    loop = jnp.arange(N, dtype=edge_index.dtype)
    src = jnp.concatenate([edge_index[0], loop])
    dst = jnp.concatenate([edge_index[1], loop])
    dsts, srcs = lax.sort_key_val(dst, src)
    srcs_pad = jnp.pad(srcs, (0, 336))
    RP = NW * NPW + NPW + 8
    rp_pad = jnp.searchsorted(dsts, jnp.arange(RP, dtype=jnp.int32)
                              ).astype(jnp.int32)
    h = jax.nn.elu(_gat(x, srcs_pad, rp_pad, W0, as0, ad0, b0, 64, True))
    h = jax.nn.elu(_gat(h, srcs_pad, rp_pad, W1, as1, ad1, b1, 256, True))
    h = jax.nn.elu(_gat(h, srcs_pad, rp_pad, W2, as2, ad2, b2, 256, True))
    h = jax.nn.elu(_gat(h, srcs_pad, rp_pad, W3, as3, ad3, b3, 256, True))
    h = jax.nn.elu(_gat(h, srcs_pad, rp_pad, W4, as4, ad4, b4, 256, False))
    G = 64
    s = jax.ops.segment_sum(h, batch, num_segments=G)
    cnt = jax.ops.segment_sum(jnp.ones((N,), dtype=h.dtype), batch,
                              num_segments=G)
    h = s / jnp.maximum(cnt, 1.0)[:, None]
    h = jax.nn.elu(h)
    return h @ linW + linb


# speedup vs baseline: 11.9183x; 1.0094x over previous
import functools

import jax
import jax.numpy as jnp
from jax import lax
from jax.experimental import pallas as pl
from jax.experimental.pallas import tpu as pltpu
from jax.experimental.pallas import tpu_sc as plsc

HEADS = 8
NW = 32
NPW = 320
CHB_BIG = 24
CHB_SMALL = 80
N_NODES = 10000


def _mm_body(x_ref, w_ref, o_ref):
    o_ref[...] = jnp.dot(x_ref[...], w_ref[...],
                         preferred_element_type=jnp.float32)


def _matmul(x, W, bm=512):
    M, K = x.shape
    _, Nc = W.shape
    Mp = ((M + bm - 1) // bm) * bm
    xp = jnp.pad(x, ((0, Mp - M), (0, 0)))
    out = pl.pallas_call(
        _mm_body,
        grid=(Mp // bm,),
        in_specs=[pl.BlockSpec((bm, K), lambda i: (i, 0)),
                  pl.BlockSpec((K, Nc), lambda i: (0, 0))],
        out_specs=pl.BlockSpec((bm, Nc), lambda i: (i, 0)),
        out_shape=jax.ShapeDtypeStruct((Mp, Nc), jnp.float32),
    )(xp, W)
    return out[:M]


@functools.partial(jax.jit, static_argnums=(5, 6))
def _agg_sc(h, srcs, als16, ald16, rp_pad, oc, mean):
    N = N_NODES
    CHB = CHB_SMALL if oc == 64 else CHB_BIG
    HOC = HEADS * oc
    VPH = oc // 16
    NV = HOC // 16
    out_cols = oc if mean else HOC
    OV = out_cols // 16
    mesh = plsc.VectorSubcoreMesh(core_axis_name="c", subcore_axis_name="s")

    @functools.partial(
        pl.kernel, mesh=mesh,
        out_type=jax.ShapeDtypeStruct((N, out_cols), jnp.float32),
        scratch_types=[
            pltpu.VMEM((NPW + 24,), jnp.int32),
            pltpu.VMEM((CHB,), jnp.int32),
            pltpu.VMEM((CHB,), jnp.int32),
            pltpu.VMEM((CHB, HOC), jnp.float32),
            pltpu.VMEM((CHB, HOC), jnp.float32),
            pltpu.VMEM((CHB, 128), jnp.float32),
            pltpu.VMEM((CHB, 128), jnp.float32),
            pltpu.VMEM((NPW * 16,), jnp.float32),
            pltpu.VMEM((128,), jnp.float32),
            pltpu.VMEM((128,), jnp.float32),
            pltpu.VMEM((HOC,), jnp.float32),
            pltpu.VMEM((128,), jnp.float32),
            pltpu.VMEM((out_cols,), jnp.float32),
            pltpu.SemaphoreType.DMA,
            pltpu.SemaphoreType.DMA,
            pltpu.SemaphoreType.DMA,
            pltpu.SemaphoreType.DMA,
            pltpu.SemaphoreType.DMA,
            pltpu.SemaphoreType.DMA,
        ])
    def k(h_hbm, srcs_hbm, als_hbm, ald_hbm, rp_hbm, out_hbm,
          rp_v, idx_v0, idx_v1, rows_v0, rows_v1, als_v0, als_v1,
          ald_v, ald_rep, exb_sc, acc, den_v, obuf,
          sem0, sem1, sem0b, sem1b, semi0, semi1):
        wid = lax.axis_index("s") * 2 + lax.axis_index("c")
        nb = wid * NPW
        pltpu.sync_copy(rp_hbm.at[pl.ds(pl.multiple_of(nb, 8), NPW + 24)],
                        rp_v)
        pltpu.sync_copy(
            ald_hbm.at[pl.ds(pl.multiple_of(nb * 16, 8), NPW * 16)], ald_v)

        def rp_at(i):
            return rp_v[pl.ds(i, 16)][0]

        e_start = rp_at(0)
        e_end = rp_at(NPW)
        zero16 = jnp.zeros((16,), jnp.float32)

        def zero_acc():
            for j in range(NV):
                acc[pl.ds(j * 16, 16)] = zero16

        zero_acc()

        def finalize(node_l):
            node_g = nb + node_l
            rpv = rp_v[pl.ds(node_l, 16)]
            deg = rpv[1] - rpv[0]

            @pl.when(deg > 0)
            def _():
                if mean:
                    for j in range(VPH):
                        s = zero16
                        for hh in range(HEADS):
                            invb = 1.0 / (den_v[pl.ds(hh * 16, 16)] + 1e-16)
                            s = s + acc[pl.ds((hh * VPH + j) * 16, 16)] \
                                * invb
                        obuf[pl.ds(j * 16, 16)] = s * (1.0 / HEADS)
                else:
                    for hh in range(HEADS):
                        invb = 1.0 / (den_v[pl.ds(hh * 16, 16)] + 1e-16)
                        for j in range(VPH):
                            c0 = (hh * VPH + j) * 16
                            obuf[pl.ds(c0, 16)] = acc[pl.ds(c0, 16)] * invb
                pltpu.sync_copy(obuf, out_hbm.at[node_g])
                zero_acc()
                for hh in range(HEADS):
                    den_v[pl.ds(hh * 16, 16)] = zero16

        for hh in range(HEADS):
            den_v[pl.ds(hh * 16, 16)] = zero16
        nxt0 = ald_v[pl.ds(0, 16)]
        for hh in range(HEADS):
            ald_rep[pl.ds(hh * 16, 16)] = nxt0[hh] + zero16
        ab0 = pl.multiple_of((e_start // 8) * 8, 8)
        nchunks = (e_end - ab0 + CHB - 1) // CHB
        bufs = ((idx_v0, rows_v0, als_v0, sem0, sem0b, semi0),
                (idx_v1, rows_v1, als_v1, sem1, sem1b, semi1))

        def fetch_idx(kk, buf):
            idx_v, rows_v, als_v, sem, semb, semi = buf
            gbase = pl.multiple_of(ab0 + kk * CHB, 8)
            pltpu.async_copy(srcs_hbm.at[pl.ds(gbase, CHB)], idx_v, semi)

        def fetch_gather(kk, buf):
            idx_v, rows_v, als_v, sem, semb, semi = buf
            gbase = pl.multiple_of(ab0 + kk * CHB, 8)
            pltpu.make_async_copy(srcs_hbm.at[pl.ds(gbase, CHB)], idx_v,
                                  semi).wait()
            pltpu.async_copy(h_hbm.at[idx_v], rows_v, sem)
            pltpu.async_copy(als_hbm.at[idx_v], als_v, semb)

        def process(kk, cur, buf):
            idx_v, rows_v, als_v, sem, semb, semi = buf
            pltpu.make_async_copy(h_hbm.at[idx_v], rows_v, sem).wait()
            pltpu.make_async_copy(als_hbm.at[idx_v], als_v, semb).wait()
            fetch_idx(kk + 2, buf)
            gbase = ab0 + kk * CHB
            lo = jnp.maximum(e_start - gbase, 0)
            hi = jnp.minimum(e_end - gbase, CHB)

            def edge_body(li, cur):
                gi = gbase + li
                seg_done = rp_at(cur + 1) <= gi

                @pl.when(seg_done)
                def _():
                    finalize(cur)
                    nxt = ald_v[pl.ds((cur + 1) * 16, 16)]
                    for hh in range(HEADS):
                        ald_rep[pl.ds(hh * 16, 16)] = nxt[hh] + zero16

                cur = jnp.where(seg_done, cur + 1, cur)
                for hh in range(HEADS):
                    hb = hh * 16
                    e = als_v[li, pl.ds(hb, 16)] + ald_rep[pl.ds(hb, 16)]
                    ex = jnp.exp(jnp.maximum(e, 0.2 * e))
                    plsc.addupdate(den_v.at[pl.ds(hb, 16)], ex)
                    exb_sc[pl.ds(hb, 16)] = ex

                @plsc.parallel_loop(0, NV, unroll=16)
                def _(i):
                    c0 = i * 16
                    hb = (i // VPH) * 16
                    plsc.addupdate(
                        acc.at[pl.ds(c0, 16)],
                        exb_sc[pl.ds(hb, 16)]
                        * rows_v[li, pl.ds(c0, 16)])

                return cur

            return lax.fori_loop(lo, hi, edge_body, cur)

        fetch_idx(0, bufs[0])
        fetch_gather(0, bufs[0])
        fetch_idx(1, bufs[1])
        fetch_gather(1, bufs[1])
        npairs = (nchunks + 1) // 2

        def pair_body(t, cur):
            cur = process(2 * t, cur, bufs[0])
            fetch_gather(2 * t + 2, bufs[0])
            cur = process(2 * t + 1, cur, bufs[1])
            fetch_gather(2 * t + 3, bufs[1])
            return cur

        cur = lax.fori_loop(0, npairs, pair_body, 0)
        for b in bufs:
            pltpu.make_async_copy(h_hbm.at[b[0]], b[1], b[3]).wait()
            pltpu.make_async_copy(als_hbm.at[b[0]], b[2], b[4]).wait()

        def dr_body(i, cur):
            @pl.when(i >= cur)
            def _():
                finalize(i)
            return cur

        lax.fori_loop(0, NPW, dr_body, cur)

    return k(h, srcs, als16, ald16, rp_pad)


def _pool_body(h_ref, b_ref, o_ref, c_ref):
    gi = pl.program_id(0)
    bb = b_ref[0, 0, :]
    pm = (bb[None, :] == jax.lax.broadcasted_iota(jnp.int32, (64, 1), 0)
          ).astype(jnp.float32)
    acc = jnp.dot(pm, h_ref[...], preferred_element_type=jnp.float32)
    cnt = jnp.broadcast_to(jnp.sum(pm, axis=1, keepdims=True), (64, 128))

    @pl.when(gi == 0)
    def _():
        o_ref[...] = jnp.zeros_like(o_ref)
        c_ref[...] = jnp.zeros_like(c_ref)

    o_ref[...] += acc
    c_ref[...] += cnt


def _pool(h, batch):
    N, D = h.shape
    NB = 10
    BL = N // NB
    out, cnt = pl.pallas_call(
        _pool_body,
        grid=(NB,),
        in_specs=[pl.BlockSpec((BL, D), lambda i: (i, 0)),
                  pl.BlockSpec((1, 1, BL), lambda i: (i, 0, 0))],
        out_specs=[pl.BlockSpec((64, D), lambda i: (0, 0)),
                   pl.BlockSpec((64, 128), lambda i: (0, 0))],
        out_shape=[jax.ShapeDtypeStruct((64, D), jnp.float32),
                   jax.ShapeDtypeStruct((64, 128), jnp.float32)],
    )(h, batch.reshape(NB, 1, BL))
    return out, cnt[:, 0]


def _gat(x, srcs_pad, rp_pad, W, a_s, a_d, b, oc, concat):
    N = x.shape[0]
    h = _matmul(x, W)
    h3 = h.reshape(N, HEADS, oc)
    al_s = jnp.sum(h3 * a_s[None], axis=-1)
    al_d = jnp.sum(h3 * a_d[None], axis=-1)
    als128 = jnp.repeat(al_s, 16, axis=1)
    ald16 = jnp.zeros((NW * NPW, 16), jnp.float32).at[:N, :8].set(
        al_d).reshape(-1)
    out = _agg_sc(h, srcs_pad, als128, ald16, rp_pad, oc, not concat)
    return out + b


def kernel(x, edge_index, batch, W0, as0, ad0, b0, W1, as1, ad1, b1, W2, as2,
           ad2, b2, W3, as3, ad3, b3, W4, as4, ad4, b4, linW, linb):
    N = x.shape[0]
    loop = jnp.arange(N, dtype=edge_index.dtype)
    src = jnp.concatenate([edge_index[0], loop])
    dst = jnp.concatenate([edge_index[1], loop])
    dsts, srcs = lax.sort_key_val(dst, src)
    srcs_pad = jnp.pad(srcs, (0, 336))
    RP = NW * NPW + NPW + 8
    rp_pad = jnp.searchsorted(dsts, jnp.arange(RP, dtype=jnp.int32)
                              ).astype(jnp.int32)
    h = jax.nn.elu(_gat(x, srcs_pad, rp_pad, W0, as0, ad0, b0, 64, True))
    h = jax.nn.elu(_gat(h, srcs_pad, rp_pad, W1, as1, ad1, b1, 256, True))
    h = jax.nn.elu(_gat(h, srcs_pad, rp_pad, W2, as2, ad2, b2, 256, True))
    h = jax.nn.elu(_gat(h, srcs_pad, rp_pad, W3, as3, ad3, b3, 256, True))
    h = jax.nn.elu(_gat(h, srcs_pad, rp_pad, W4, as4, ad4, b4, 256, False))
    s, cnt = _pool(h, batch)
    h = s / jnp.maximum(cnt, 1.0)[:, None]
    h = jax.nn.elu(h)
    return h @ linW + linb
